# Initial kernel scaffold; baseline (speedup 1.0000x reference)
import jax
import jax.numpy as jnp
from jax.experimental import pallas as pl


def kernel(idx, table):
    raise NotImplementedError("write your pallas kernel here")


# speedup vs baseline: 1.5594x; 1.5594x over previous
import functools

import jax
import jax.numpy as jnp
from jax import lax
from jax.experimental import pallas as pl
from jax.experimental.pallas import tpu as pltpu, tpu_sc as plsc

NUM_ROWS = 16384 * 26
DIM = 32
NW = 32
PER_W = NUM_ROWS // NW
CHUNK = 128
GROUP = 8
N_CHUNKS = PER_W // CHUNK
N_GROUPS = N_CHUNKS // GROUP
GROUP_ROWS = GROUP * CHUNK


def _gather_grid(idx3, table):
    mesh = plsc.VectorSubcoreMesh(core_axis_name="c", subcore_axis_name="s")

    @functools.partial(
        pl.kernel,
        mesh=mesh,
        out_type=jax.ShapeDtypeStruct((NUM_ROWS, DIM), jnp.float32),
        scratch_types=[
            pltpu.VMEM((N_CHUNKS, CHUNK), jnp.int32),
            pltpu.VMEM((GROUP_ROWS, DIM), jnp.float32),
            pltpu.SemaphoreType.DMA,
        ],
        compiler_params=pltpu.CompilerParams(use_tc_tiling_on_sc=False),
    )
    def k(idx_hbm, table_hbm, out_hbm, idx_v, rows_v, sem):
        wid = lax.axis_index("s") * 2 + lax.axis_index("c")
        base = wid * PER_W
        pltpu.sync_copy(idx_hbm.at[wid], idx_v)

        def body(g, carry):
            waits = []
            for b in range(GROUP):
                cp = pltpu.make_async_copy(
                    table_hbm.at[idx_v.at[g * GROUP + b]],
                    rows_v.at[pl.ds(b * CHUNK, CHUNK)],
                    sem,
                )
                cp.start()
                waits.append(cp)
            for cp in waits:
                cp.wait()
            pltpu.sync_copy(
                rows_v, out_hbm.at[pl.ds(base + g * GROUP_ROWS, GROUP_ROWS)]
            )
            return carry

        lax.fori_loop(0, N_GROUPS, body, 0)

    return k(idx3, table)


def kernel(idx, table):
    idx3 = idx.reshape(NW, N_CHUNKS, CHUNK).astype(jnp.int32)
    out = _gather_grid(idx3, table)
    return out.reshape(idx.shape[0], idx.shape[1], DIM)


# speedup vs baseline: 1.5720x; 1.0081x over previous
import functools

import jax
import jax.numpy as jnp
from jax import lax
from jax.experimental import pallas as pl
from jax.experimental.pallas import tpu as pltpu, tpu_sc as plsc

NUM_ROWS = 16384 * 26
DIM = 32
NW = 32
PER_W = NUM_ROWS // NW
CHUNK = 128
GROUP = 8
N_CHUNKS = PER_W // CHUNK
N_GROUPS = N_CHUNKS // GROUP
GROUP_ROWS = GROUP * CHUNK
NBUF = 3


def _gather_grid(idx3, table):
    mesh = plsc.VectorSubcoreMesh(core_axis_name="c", subcore_axis_name="s")

    @functools.partial(
        pl.kernel,
        mesh=mesh,
        out_type=jax.ShapeDtypeStruct((NUM_ROWS, DIM), jnp.float32),
        scratch_types=[
            pltpu.VMEM((N_CHUNKS, CHUNK), jnp.int32),
            pltpu.VMEM((NBUF, GROUP_ROWS, DIM), jnp.float32),
            pltpu.SemaphoreType.DMA,
            pltpu.SemaphoreType.DMA,
        ],
        compiler_params=pltpu.CompilerParams(use_tc_tiling_on_sc=False),
    )
    def k(idx_hbm, table_hbm, out_hbm, idx_v, rows_v, sem_in, sem_out):
        wid = lax.axis_index("s") * 2 + lax.axis_index("c")
        base = wid * PER_W
        pltpu.sync_copy(idx_hbm.at[wid], idx_v)

        def start_gathers(g):
            slot = g % NBUF
            waits = []
            for b in range(GROUP):
                cp = pltpu.make_async_copy(
                    table_hbm.at[idx_v.at[g * GROUP + b]],
                    rows_v.at[slot, pl.ds(b * CHUNK, CHUNK)],
                    sem_in,
                )
                cp.start()
                waits.append(cp)
            return waits

        def make_out(g):
            return pltpu.make_async_copy(
                rows_v.at[g % NBUF],
                out_hbm.at[pl.ds(base + g * GROUP_ROWS, GROUP_ROWS)],
                sem_out,
            )

        in_flight = {0: start_gathers(0)}
        out_flight = {}
        for g in range(N_GROUPS):
            if g + 1 < N_GROUPS:
                prev = g + 1 - NBUF
                if prev in out_flight:
                    out_flight.pop(prev).wait()
                in_flight[g + 1] = start_gathers(g + 1)
            for cp in in_flight.pop(g):
                cp.wait()
            cp_out = make_out(g)
            cp_out.start()
            out_flight[g] = cp_out
        for g in sorted(out_flight):
            out_flight.pop(g).wait()

    return k(idx3, table)


def kernel(idx, table):
    idx3 = idx.reshape(NW, N_CHUNKS, CHUNK).astype(jnp.int32)
    out = _gather_grid(idx3, table)
    return out.reshape(idx.shape[0], idx.shape[1], DIM)
